# Initial kernel scaffold; baseline (speedup 1.0000x reference)
import functools

import jax
import jax.numpy as jnp
from jax import lax
from jax.experimental import pallas as pl
from jax.experimental.pallas import tpu as pltpu
from jax.experimental.pallas import tpu_sc as plsc

N_NODES = 10000
N_EDGES = 320000
D = 128

NC = 2
NS = 16
NW = NC * NS
CHUNK = 128
EPW = -(-N_EDGES // NW)
NCHUNK = -(-EPW // CHUNK)
EPW_PAD = NCHUNK * CHUNK
NPAD = 10240
ROWS_PT = NPAD // NS
DEG_W = 16


def _make_sc_agg(compute_deg):
    mesh = plsc.VectorSubcoreMesh(core_axis_name="c", subcore_axis_name="s")

    out_type = [jax.ShapeDtypeStruct((NC, NPAD, D), jnp.float32)]
    if compute_deg:
        out_type.append(jax.ShapeDtypeStruct((NC, NPAD, DEG_W), jnp.float32))

    scratch = [
        pltpu.VMEM((NCHUNK, CHUNK), jnp.int32),
        pltpu.VMEM((NCHUNK, CHUNK), jnp.int32),
        pltpu.VMEM((CHUNK, D), jnp.float32),
        pltpu.VMEM((CHUNK, DEG_W), jnp.float32),
        pltpu.VMEM((16, D), jnp.float32),
        pltpu.VMEM((16, DEG_W), jnp.float32),
        pltpu.VMEM_SHARED((NPAD, D), jnp.float32),
        pltpu.VMEM_SHARED((NPAD, DEG_W), jnp.float32),
        pltpu.SemaphoreType.DMA,
    ]

    @functools.partial(
        pl.kernel,
        out_type=tuple(out_type) if compute_deg else out_type[0],
        mesh=mesh,
        scratch_types=scratch,
    )
    def sc_agg(h_hbm, src_hbm, dst_hbm, *rest):
        if compute_deg:
            agg_hbm, deg_hbm = rest[0], rest[1]
            scratches = rest[2:]
        else:
            agg_hbm = rest[0]
            deg_hbm = None
            scratches = rest[1:]
        src_v, dst_v, gbuf, ones_v, zrow, zrow_d, acc, dacc, gsem = scratches

        c = lax.axis_index("c")
        s = lax.axis_index("s")
        wid = s * NC + c
        r0 = s * ROWS_PT

        zero16 = jnp.zeros((16,), jnp.float32)
        for r in range(16):
            for q in range(D // 16):
                zrow[r, pl.ds(q * 16, 16)] = zero16
            zrow_d[r, pl.ds(0, 16)] = zero16
        if compute_deg:
            one16 = jnp.ones((16,), jnp.float32)
            for r in range(CHUNK):
                ones_v[r, pl.ds(0, 16)] = one16

        def _zero(i, _):
            pltpu.sync_copy(zrow, acc.at[pl.ds(r0 + i * 16, 16)])
            pltpu.sync_copy(zrow_d, dacc.at[pl.ds(r0 + i * 16, 16)])
            return ()

        lax.fori_loop(0, ROWS_PT // 16, _zero, ())
        plsc.subcore_barrier()

        pltpu.sync_copy(src_hbm.at[wid], src_v)
        pltpu.sync_copy(dst_hbm.at[wid], dst_v)

        def _edge_chunk(j, _):
            pltpu.async_copy(h_hbm.at[src_v.at[j]], gbuf, gsem).wait()
            pltpu.sync_copy(gbuf, acc.at[dst_v.at[j]], add=True)
            if compute_deg:
                pltpu.sync_copy(ones_v, dacc.at[dst_v.at[j]], add=True)
            return ()

        lax.fori_loop(0, NCHUNK, _edge_chunk, ())
        plsc.subcore_barrier()

        pltpu.sync_copy(acc.at[pl.ds(r0, ROWS_PT)],
                        agg_hbm.at[c, pl.ds(r0, ROWS_PT)])
        if compute_deg:
            pltpu.sync_copy(dacc.at[pl.ds(r0, ROWS_PT)],
                            deg_hbm.at[c, pl.ds(r0, ROWS_PT)])

    return sc_agg


_sc_agg_with_deg = _make_sc_agg(True)
_sc_agg_no_deg = _make_sc_agg(False)


ROW_BLK = 400
N_BLKS = N_NODES // ROW_BLK


def _combine_body(h_ref, agg_ref, deg_ref, ws_ref, wn_ref, b_ref, o_ref,
                  *, relu):
    agg = agg_ref[0] + agg_ref[1]
    deg = deg_ref[0][:, :1] + deg_ref[1][:, :1]
    hn = agg * (1.0 / jnp.maximum(deg, 1.0))
    h = h_ref[...]
    out = (jnp.dot(h, ws_ref[...], preferred_element_type=jnp.float32)
           + jnp.dot(hn, wn_ref[...], preferred_element_type=jnp.float32)
           + b_ref[...])
    if relu:
        out = jnp.maximum(out, 0.0)
    o_ref[...] = out


def _make_combine(relu):
    return pl.pallas_call(
        functools.partial(_combine_body, relu=relu),
        grid=(N_BLKS,),
        in_specs=[
            pl.BlockSpec((ROW_BLK, D), lambda i: (i, 0)),
            pl.BlockSpec((NC, ROW_BLK, D), lambda i: (0, i, 0)),
            pl.BlockSpec((NC, ROW_BLK, DEG_W), lambda i: (0, i, 0)),
            pl.BlockSpec((D, D), lambda i: (0, 0)),
            pl.BlockSpec((D, D), lambda i: (0, 0)),
            pl.BlockSpec((1, D), lambda i: (0, 0)),
        ],
        out_specs=pl.BlockSpec((ROW_BLK, D), lambda i: (i, 0)),
        out_shape=jax.ShapeDtypeStruct((N_NODES, D), jnp.float32),
    )


_combine_relu = _make_combine(True)
_combine_plain = _make_combine(False)


@jax.jit
def kernel(input_features, edge_index, W_self1, W_neigh1, b1,
           W_self2, W_neigh2, b2):
    src = edge_index[0].astype(jnp.int32)
    dst = edge_index[1].astype(jnp.int32)
    pad = NW * EPW_PAD - N_EDGES
    src = jnp.concatenate([src, jnp.zeros((pad,), jnp.int32)])
    dst = jnp.concatenate([dst, jnp.full((pad,), N_NODES, jnp.int32)])
    src_t = src.reshape(NW, NCHUNK, CHUNK)
    dst_t = dst.reshape(NW, NCHUNK, CHUNK)
    b1r = b1.reshape(1, D)
    b2r = b2.reshape(1, D)

    agg1, deg = _sc_agg_with_deg(input_features, src_t, dst_t)
    h1 = _combine_relu(input_features, agg1, deg, W_self1, W_neigh1, b1r)
    agg2 = _sc_agg_no_deg(h1, src_t, dst_t)
    return _combine_plain(h1, agg2, deg, W_self2, W_neigh2, b2r)


# speedup vs baseline: 3.0773x; 3.0773x over previous
import functools

import jax
import jax.numpy as jnp
from jax import lax
from jax.experimental import pallas as pl
from jax.experimental.pallas import tpu as pltpu
from jax.experimental.pallas import tpu_sc as plsc

N_NODES = 10000
N_EDGES = 320000
D = 128

NC = 2
NS = 16
NW = NC * NS
CHUNK = 128
BLK = 8
NBLK = 10
NCHUNK = BLK * NBLK
EPW_PAD = NCHUNK * CHUNK
NPAD = 10240
ROWS_PT = NPAD // NS


def _make_sc_agg(compute_deg):
    mesh = plsc.VectorSubcoreMesh(core_axis_name="c", subcore_axis_name="s")

    out_type = [jax.ShapeDtypeStruct((NC, NPAD, D), jnp.float32)]
    scratch = [
        pltpu.VMEM((BLK, CHUNK), jnp.int32),
        pltpu.VMEM((BLK, CHUNK), jnp.int32),
        pltpu.VMEM((CHUNK, D), jnp.float32),
        pltpu.VMEM((16, D), jnp.float32),
        pltpu.VMEM_SHARED((NPAD, D), jnp.float32),
        pltpu.SemaphoreType.DMA,
    ]
    if compute_deg:
        out_type.append(jax.ShapeDtypeStruct((NW, NPAD), jnp.float32))
        scratch.insert(4, pltpu.VMEM((NPAD,), jnp.float32))

    @functools.partial(
        pl.kernel,
        out_type=tuple(out_type) if compute_deg else out_type[0],
        mesh=mesh,
        scratch_types=scratch,
        compiler_params=pltpu.CompilerParams(needs_layout_passes=False),
    )
    def sc_agg(h_hbm, src_hbm, dst_hbm, *rest):
        if compute_deg:
            agg_hbm, deg_hbm = rest[0], rest[1]
            src_v, dst_v, gbuf, zrow, deg_v, acc, gsem = rest[2:]
        else:
            agg_hbm = rest[0]
            src_v, dst_v, gbuf, zrow, acc, gsem = rest[1:]

        c = lax.axis_index("c")
        s = lax.axis_index("s")
        wid = s * NC + c
        r0 = s * ROWS_PT

        zero16 = jnp.zeros((16,), jnp.float32)
        one16 = jnp.ones((16,), jnp.float32)
        for r in range(16):
            for q in range(D // 16):
                zrow[r, pl.ds(q * 16, 16)] = zero16

        if compute_deg:
            def _zero_deg(i, _):
                deg_v[pl.ds(i * 16, 16)] = zero16
                return ()

            lax.fori_loop(0, NPAD // 16, _zero_deg, ())

        def _zero(i, _):
            pltpu.sync_copy(zrow, acc.at[pl.ds(r0 + i * 16, 16)])
            return ()

        lax.fori_loop(0, ROWS_PT // 16, _zero, ())
        plsc.subcore_barrier()

        def _blk(bi, _):
            pltpu.sync_copy(src_hbm.at[wid, pl.ds(bi * BLK, BLK)], src_v)
            pltpu.sync_copy(dst_hbm.at[wid, pl.ds(bi * BLK, BLK)], dst_v)

            def _chunk(j, _):
                pltpu.async_copy(h_hbm.at[src_v.at[j]], gbuf, gsem).wait()
                pltpu.sync_copy(gbuf, acc.at[dst_v.at[j]], add=True)
                if compute_deg:
                    for q in range(CHUNK // 16):
                        idx16 = dst_v[j, pl.ds(q * 16, 16)]
                        plsc.addupdate_scatter(deg_v, [idx16], one16)
                return ()

            lax.fori_loop(0, BLK, _chunk, ())
            return ()

        lax.fori_loop(0, NBLK, _blk, ())
        plsc.subcore_barrier()

        pltpu.sync_copy(acc.at[pl.ds(r0, ROWS_PT)],
                        agg_hbm.at[c, pl.ds(r0, ROWS_PT)])
        if compute_deg:
            pltpu.sync_copy(deg_v, deg_hbm.at[wid])

    return sc_agg


_sc_agg_with_deg = _make_sc_agg(True)
_sc_agg_no_deg = _make_sc_agg(False)


ROW_BLK = 400
N_BLKS = N_NODES // ROW_BLK


def _combine_body(h_ref, agg_ref, deg_ref, ws_ref, wn_ref, b_ref, o_ref,
                  *, relu):
    agg = agg_ref[0] + agg_ref[1]
    deg = jnp.sum(deg_ref[...], axis=1, keepdims=True)
    hn = agg * (1.0 / jnp.maximum(deg, 1.0))
    h = h_ref[...]
    out = (jnp.dot(h, ws_ref[...], preferred_element_type=jnp.float32)
           + jnp.dot(hn, wn_ref[...], preferred_element_type=jnp.float32)
           + b_ref[...])
    if relu:
        out = jnp.maximum(out, 0.0)
    o_ref[...] = out


def _make_combine(relu):
    return pl.pallas_call(
        functools.partial(_combine_body, relu=relu),
        grid=(N_BLKS,),
        in_specs=[
            pl.BlockSpec((ROW_BLK, D), lambda i: (i, 0)),
            pl.BlockSpec((NC, ROW_BLK, D), lambda i: (0, i, 0)),
            pl.BlockSpec((ROW_BLK, NW), lambda i: (i, 0)),
            pl.BlockSpec((D, D), lambda i: (0, 0)),
            pl.BlockSpec((D, D), lambda i: (0, 0)),
            pl.BlockSpec((1, D), lambda i: (0, 0)),
        ],
        out_specs=pl.BlockSpec((ROW_BLK, D), lambda i: (i, 0)),
        out_shape=jax.ShapeDtypeStruct((N_NODES, D), jnp.float32),
    )


_combine_relu = _make_combine(True)
_combine_plain = _make_combine(False)


@jax.jit
def kernel(input_features, edge_index, W_self1, W_neigh1, b1,
           W_self2, W_neigh2, b2):
    src = edge_index[0].astype(jnp.int32)
    dst = edge_index[1].astype(jnp.int32)
    pad = NW * EPW_PAD - N_EDGES
    src = jnp.concatenate([src, jnp.zeros((pad,), jnp.int32)])
    dst = jnp.concatenate([dst, jnp.full((pad,), N_NODES, jnp.int32)])
    src_t = src.reshape(NW, NCHUNK, CHUNK)
    dst_t = dst.reshape(NW, NCHUNK, CHUNK)
    b1r = b1.reshape(1, D)
    b2r = b2.reshape(1, D)

    agg1, deg = _sc_agg_with_deg(input_features, src_t, dst_t)
    deg_t = deg.T
    h1 = _combine_relu(input_features, agg1, deg_t, W_self1, W_neigh1, b1r)
    agg2 = _sc_agg_no_deg(h1, src_t, dst_t)
    return _combine_plain(h1, agg2, deg_t, W_self2, W_neigh2, b2r)


# speedup vs baseline: 3.3683x; 1.0946x over previous
import functools

import jax
import jax.numpy as jnp
from jax import lax
from jax.experimental import pallas as pl
from jax.experimental.pallas import tpu as pltpu
from jax.experimental.pallas import tpu_sc as plsc

N_NODES = 10000
N_EDGES = 320000
D = 128

NC = 2
NS = 16
NW = NC * NS
CHUNK = 128
BLK = 8
NBLK = 10
NCHUNK = BLK * NBLK
NBUF = 2
EPW_PAD = NCHUNK * CHUNK
NPAD = 10240
ROWS_PT = NPAD // NS


def _make_sc_agg(compute_deg):
    mesh = plsc.VectorSubcoreMesh(core_axis_name="c", subcore_axis_name="s")

    out_type = [jax.ShapeDtypeStruct((NC, NPAD, D), jnp.float32)]
    scratch = [
        pltpu.VMEM((BLK, CHUNK), jnp.int32),
        pltpu.VMEM((BLK, CHUNK), jnp.int32),
        pltpu.VMEM((NBUF, CHUNK, D), jnp.float32),
        pltpu.VMEM((16, D), jnp.float32),
        pltpu.VMEM_SHARED((NPAD, D), jnp.float32),
    ] + [pltpu.SemaphoreType.DMA] * NBUF
    if compute_deg:
        out_type.append(jax.ShapeDtypeStruct((NW, NPAD), jnp.float32))
        scratch.insert(4, pltpu.VMEM((NPAD,), jnp.float32))

    @functools.partial(
        pl.kernel,
        out_type=tuple(out_type) if compute_deg else out_type[0],
        mesh=mesh,
        scratch_types=scratch,
        compiler_params=pltpu.CompilerParams(needs_layout_passes=False),
    )
    def sc_agg(h_hbm, src_hbm, dst_hbm, *rest):
        if compute_deg:
            agg_hbm, deg_hbm = rest[0], rest[1]
            src_v, dst_v, gbuf, zrow, deg_v, acc = rest[2:8]
            gsems = rest[8:]
        else:
            agg_hbm = rest[0]
            src_v, dst_v, gbuf, zrow, acc = rest[1:6]
            gsems = rest[6:]

        c = lax.axis_index("c")
        s = lax.axis_index("s")
        wid = s * NC + c
        r0 = s * ROWS_PT

        zero16 = jnp.zeros((16,), jnp.float32)
        one16 = jnp.ones((16,), jnp.float32)
        for r in range(16):
            for q in range(D // 16):
                zrow[r, pl.ds(q * 16, 16)] = zero16

        if compute_deg:
            def _zero_deg(i, _):
                deg_v[pl.ds(i * 16, 16)] = zero16
                return ()

            lax.fori_loop(0, NPAD // 16, _zero_deg, ())

        def _zero(i, _):
            pltpu.sync_copy(zrow, acc.at[pl.ds(r0 + i * 16, 16)])
            return ()

        lax.fori_loop(0, ROWS_PT // 16, _zero, ())
        plsc.subcore_barrier()

        def _gather_start(j, b):
            pltpu.make_async_copy(
                h_hbm.at[src_v.at[j]], gbuf.at[b], gsems[b]).start()

        def _scatter(j, b):
            pltpu.make_async_copy(
                h_hbm.at[src_v.at[j]], gbuf.at[b], gsems[b]).wait()
            pltpu.sync_copy(gbuf.at[b], acc.at[dst_v.at[j]], add=True)
            if compute_deg:
                for q in range(CHUNK // 16):
                    idx16 = dst_v[j, pl.ds(q * 16, 16)]
                    plsc.addupdate_scatter(deg_v, [idx16], one16)

        pltpu.sync_copy(src_hbm.at[wid, pl.ds(0, BLK)], src_v)
        pltpu.sync_copy(dst_hbm.at[wid, pl.ds(0, BLK)], dst_v)
        for b in range(NBUF):
            _gather_start(b, b)

        def _blk(bi, _):
            def _grp(g, _):
                for b in range(NBUF):
                    j = g * NBUF + b
                    _scatter(j, b)
                    _gather_start(j + NBUF, b)
                return ()

            lax.fori_loop(0, (BLK - NBUF) // NBUF, _grp, ())

            for b in range(NBUF):
                _scatter(BLK - NBUF + b, b)

            @pl.when(bi + 1 < NBLK)
            def _():
                pltpu.sync_copy(
                    src_hbm.at[wid, pl.ds((bi + 1) * BLK, BLK)], src_v)
                pltpu.sync_copy(
                    dst_hbm.at[wid, pl.ds((bi + 1) * BLK, BLK)], dst_v)
                for b in range(NBUF):
                    _gather_start(b, b)

            return ()

        lax.fori_loop(0, NBLK, _blk, ())
        plsc.subcore_barrier()

        pltpu.sync_copy(acc.at[pl.ds(r0, ROWS_PT)],
                        agg_hbm.at[c, pl.ds(r0, ROWS_PT)])
        if compute_deg:
            pltpu.sync_copy(deg_v, deg_hbm.at[wid])

    return sc_agg


_sc_agg_with_deg = _make_sc_agg(True)
_sc_agg_no_deg = _make_sc_agg(False)


ROW_BLK = 400
N_BLKS = N_NODES // ROW_BLK


def _combine_body(h_ref, agg_ref, deg_ref, ws_ref, wn_ref, b_ref, o_ref,
                  *, relu):
    agg = agg_ref[0] + agg_ref[1]
    deg = jnp.sum(deg_ref[...], axis=1, keepdims=True)
    hn = agg * (1.0 / jnp.maximum(deg, 1.0))
    h = h_ref[...]
    out = (jnp.dot(h, ws_ref[...], preferred_element_type=jnp.float32)
           + jnp.dot(hn, wn_ref[...], preferred_element_type=jnp.float32)
           + b_ref[...])
    if relu:
        out = jnp.maximum(out, 0.0)
    o_ref[...] = out


def _make_combine(relu):
    return pl.pallas_call(
        functools.partial(_combine_body, relu=relu),
        grid=(N_BLKS,),
        in_specs=[
            pl.BlockSpec((ROW_BLK, D), lambda i: (i, 0)),
            pl.BlockSpec((NC, ROW_BLK, D), lambda i: (0, i, 0)),
            pl.BlockSpec((ROW_BLK, NW), lambda i: (i, 0)),
            pl.BlockSpec((D, D), lambda i: (0, 0)),
            pl.BlockSpec((D, D), lambda i: (0, 0)),
            pl.BlockSpec((1, D), lambda i: (0, 0)),
        ],
        out_specs=pl.BlockSpec((ROW_BLK, D), lambda i: (i, 0)),
        out_shape=jax.ShapeDtypeStruct((N_NODES, D), jnp.float32),
    )


_combine_relu = _make_combine(True)
_combine_plain = _make_combine(False)


@jax.jit
def kernel(input_features, edge_index, W_self1, W_neigh1, b1,
           W_self2, W_neigh2, b2):
    src = edge_index[0].astype(jnp.int32)
    dst = edge_index[1].astype(jnp.int32)
    pad = NW * EPW_PAD - N_EDGES
    src = jnp.concatenate([src, jnp.zeros((pad,), jnp.int32)])
    dst = jnp.concatenate([dst, jnp.full((pad,), N_NODES, jnp.int32)])
    src_t = src.reshape(NW, NCHUNK, CHUNK)
    dst_t = dst.reshape(NW, NCHUNK, CHUNK)
    b1r = b1.reshape(1, D)
    b2r = b2.reshape(1, D)

    agg1, deg = _sc_agg_with_deg(input_features, src_t, dst_t)
    deg_t = deg.T
    h1 = _combine_relu(input_features, agg1, deg_t, W_self1, W_neigh1, b1r)
    agg2 = _sc_agg_no_deg(h1, src_t, dst_t)
    return _combine_plain(h1, agg2, deg_t, W_self2, W_neigh2, b2r)
